# baseline (speedup 1.0000x reference)
import jax
import jax.numpy as jnp
from jax.experimental import pallas as pl
from jax.experimental.pallas import tpu as pltpu

_B = 64
_C = 384
_PB = 56
_NBUF = 5


def _film_ring_body(y_ref, gt_ref, bt_ref, xt_ref, o_ref,
                    gvm, bvm, xbuf, obuf, insem, outsem):
    p = xt_ref.shape[0]
    nchunks = p // _PB

    for k in range(_NBUF):
        pltpu.make_async_copy(
            xt_ref.at[pl.ds(k * _PB, _PB)], xbuf.at[k], insem.at[k]
        ).start()

    for i in range(_B):
        row = y_ref[i]
        gvm[i, :] = gt_ref[row, :]
        bvm[i, :] = bt_ref[row, :]
    g = gvm[...]
    b = bvm[...]

    for j in range(nchunks):
        slot = j % _NBUF
        pltpu.make_async_copy(
            xt_ref.at[pl.ds(j * _PB, _PB)], xbuf.at[slot], insem.at[slot]
        ).wait()

        if j >= _NBUF:
            pltpu.make_async_copy(
                obuf.at[slot], o_ref.at[pl.ds(0, _PB)], outsem.at[slot]
            ).wait()

        obuf[slot] = xbuf[slot] * g + b

        pltpu.make_async_copy(
            obuf.at[slot], o_ref.at[pl.ds(j * _PB, _PB)], outsem.at[slot]
        ).start(priority=1)

        if j + _NBUF < nchunks:
            pltpu.make_async_copy(
                xt_ref.at[pl.ds((j + _NBUF) * _PB, _PB)],
                xbuf.at[slot],
                insem.at[slot],
            ).start()

    for k in range(_NBUF):
        pltpu.make_async_copy(
            obuf.at[k], o_ref.at[pl.ds(0, _PB)], outsem.at[k]
        ).wait()


def _film_planes(y, gt, bt, xt):
    p, bsz, c = xt.shape
    return pl.pallas_call(
        _film_ring_body,
        in_specs=[
            pl.BlockSpec(memory_space=pltpu.SMEM),
            pl.BlockSpec(memory_space=pltpu.VMEM),
            pl.BlockSpec(memory_space=pltpu.VMEM),
            pl.BlockSpec(memory_space=pltpu.HBM),
        ],
        out_specs=pl.BlockSpec(memory_space=pltpu.HBM),
        out_shape=jax.ShapeDtypeStruct((p, bsz, c), xt.dtype),
        scratch_shapes=[
            pltpu.VMEM((_B, _C), jnp.float32),
            pltpu.VMEM((_B, _C), jnp.float32),
            pltpu.VMEM((_NBUF, _PB, _B, _C), jnp.float32),
            pltpu.VMEM((_NBUF, _PB, _B, _C), jnp.float32),
            pltpu.SemaphoreType.DMA((_NBUF,)),
            pltpu.SemaphoreType.DMA((_NBUF,)),
        ],
        compiler_params=pltpu.CompilerParams(
            vmem_limit_bytes=60 * 1024 * 1024,
        ),
    )(y, gt, bt, xt)


def kernel(x, y, gammas_table, betas_table):
    bsz, c, h, w = x.shape
    xt = jnp.transpose(x, (2, 3, 0, 1)).reshape(h * w, bsz, c)
    ot = _film_planes(y.astype(jnp.int32), gammas_table, betas_table, xt)
    out = jnp.transpose(ot.reshape(h, w, bsz, c), (2, 3, 0, 1))
    return (out, y)


# speedup vs baseline: 1.0136x; 1.0136x over previous
import jax
import jax.numpy as jnp
from jax.experimental import pallas as pl
from jax.experimental.pallas import tpu as pltpu

_B = 64
_C = 384
_PBMAX = 98
_NBUF = 3
_CHUNKS = (28, 42, 84, 98, 98, 98, 98, 98, 84, 42, 14)
_OFFS = tuple(sum(_CHUNKS[:i]) for i in range(len(_CHUNKS)))


def _film_ring_body(y_ref, gt_ref, bt_ref, xt_ref, o_ref,
                    gvm, bvm, xbuf, obuf, insem, outsem):
    nchunks = len(_CHUNKS)

    def in_copy(j, slot):
        sz = _CHUNKS[j]
        return pltpu.make_async_copy(
            xt_ref.at[pl.ds(_OFFS[j], sz)],
            xbuf.at[slot, pl.ds(0, sz)],
            insem.at[slot],
        )

    def out_copy(j, slot):
        sz = _CHUNKS[j]
        return pltpu.make_async_copy(
            obuf.at[slot, pl.ds(0, sz)],
            o_ref.at[pl.ds(_OFFS[j], sz)],
            outsem.at[slot],
        )

    for k in range(_NBUF):
        in_copy(k, k).start()

    for i in range(_B):
        row = y_ref[i]
        gvm[i, :] = gt_ref[row, :]
        bvm[i, :] = bt_ref[row, :]
    g = gvm[...]
    b = bvm[...]

    for j in range(nchunks):
        slot = j % _NBUF
        sz = _CHUNKS[j]
        in_copy(j, slot).wait()

        if j >= _NBUF:
            out_copy(j - _NBUF, slot).wait()

        obuf[slot, :sz] = xbuf[slot, :sz] * g + b

        out_copy(j, slot).start(priority=1)

        if j + _NBUF < nchunks:
            in_copy(j + _NBUF, slot).start()

    for k in range(_NBUF):
        slot = (nchunks - _NBUF + k) % _NBUF
        out_copy(nchunks - _NBUF + k, slot).wait()


def _film_planes(y, gt, bt, xt):
    p, bsz, c = xt.shape
    return pl.pallas_call(
        _film_ring_body,
        in_specs=[
            pl.BlockSpec(memory_space=pltpu.SMEM),
            pl.BlockSpec(memory_space=pltpu.VMEM),
            pl.BlockSpec(memory_space=pltpu.VMEM),
            pl.BlockSpec(memory_space=pltpu.HBM),
        ],
        out_specs=pl.BlockSpec(memory_space=pltpu.HBM),
        out_shape=jax.ShapeDtypeStruct((p, bsz, c), xt.dtype),
        scratch_shapes=[
            pltpu.VMEM((_B, _C), jnp.float32),
            pltpu.VMEM((_B, _C), jnp.float32),
            pltpu.VMEM((_NBUF, _PBMAX, _B, _C), jnp.float32),
            pltpu.VMEM((_NBUF, _PBMAX, _B, _C), jnp.float32),
            pltpu.SemaphoreType.DMA((_NBUF,)),
            pltpu.SemaphoreType.DMA((_NBUF,)),
        ],
        compiler_params=pltpu.CompilerParams(
            vmem_limit_bytes=60 * 1024 * 1024,
        ),
    )(y, gt, bt, xt)


def kernel(x, y, gammas_table, betas_table):
    bsz, c, h, w = x.shape
    xt = jnp.transpose(x, (2, 3, 0, 1)).reshape(h * w, bsz, c)
    ot = _film_planes(y.astype(jnp.int32), gammas_table, betas_table, xt)
    out = jnp.transpose(ot.reshape(h, w, bsz, c), (2, 3, 0, 1))
    return (out, y)


# speedup vs baseline: 1.0158x; 1.0022x over previous
import jax
import jax.numpy as jnp
from jax.experimental import pallas as pl
from jax.experimental.pallas import tpu as pltpu

_B = 64
_C = 384
_PBMAX = 98
_NBUF = 3
_CHUNKS = (14, 28, 42, 98, 98, 98, 98, 98, 98, 70, 28, 14)
_OFFS = tuple(sum(_CHUNKS[:i]) for i in range(len(_CHUNKS)))


def _film_ring_body(y_ref, gt_ref, bt_ref, xt_ref, o_ref,
                    gvm, bvm, xbuf, obuf, insem, outsem):
    nchunks = len(_CHUNKS)

    def in_copy(j, slot):
        sz = _CHUNKS[j]
        return pltpu.make_async_copy(
            xt_ref.at[pl.ds(_OFFS[j], sz)],
            xbuf.at[slot, pl.ds(0, sz)],
            insem.at[slot],
        )

    def out_copy(j, slot):
        sz = _CHUNKS[j]
        return pltpu.make_async_copy(
            obuf.at[slot, pl.ds(0, sz)],
            o_ref.at[pl.ds(_OFFS[j], sz)],
            outsem.at[slot],
        )

    for k in range(_NBUF):
        in_copy(k, k).start()

    for i in range(_B):
        row = y_ref[i]
        gvm[i, :] = gt_ref[row, :]
        bvm[i, :] = bt_ref[row, :]
    g = gvm[...]
    b = bvm[...]

    for j in range(nchunks):
        slot = j % _NBUF
        sz = _CHUNKS[j]
        in_copy(j, slot).wait()

        if j >= _NBUF:
            out_copy(j - _NBUF, slot).wait()

        obuf[slot, :sz] = xbuf[slot, :sz] * g + b

        out_copy(j, slot).start(priority=1)

        if j + _NBUF < nchunks:
            in_copy(j + _NBUF, slot).start()

    for k in range(_NBUF):
        slot = (nchunks - _NBUF + k) % _NBUF
        out_copy(nchunks - _NBUF + k, slot).wait()


def _film_planes(y, gt, bt, xt):
    p, bsz, c = xt.shape
    return pl.pallas_call(
        _film_ring_body,
        in_specs=[
            pl.BlockSpec(memory_space=pltpu.SMEM),
            pl.BlockSpec(memory_space=pltpu.VMEM),
            pl.BlockSpec(memory_space=pltpu.VMEM),
            pl.BlockSpec(memory_space=pltpu.HBM),
        ],
        out_specs=pl.BlockSpec(memory_space=pltpu.HBM),
        out_shape=jax.ShapeDtypeStruct((p, bsz, c), xt.dtype),
        scratch_shapes=[
            pltpu.VMEM((_B, _C), jnp.float32),
            pltpu.VMEM((_B, _C), jnp.float32),
            pltpu.VMEM((_NBUF, _PBMAX, _B, _C), jnp.float32),
            pltpu.VMEM((_NBUF, _PBMAX, _B, _C), jnp.float32),
            pltpu.SemaphoreType.DMA((_NBUF,)),
            pltpu.SemaphoreType.DMA((_NBUF,)),
        ],
        compiler_params=pltpu.CompilerParams(
            vmem_limit_bytes=60 * 1024 * 1024,
        ),
    )(y, gt, bt, xt)


def kernel(x, y, gammas_table, betas_table):
    bsz, c, h, w = x.shape
    xt = jnp.transpose(x, (2, 3, 0, 1)).reshape(h * w, bsz, c)
    ot = _film_planes(y.astype(jnp.int32), gammas_table, betas_table, xt)
    out = jnp.transpose(ot.reshape(h, w, bsz, c), (2, 3, 0, 1))
    return (out, y)
